# baseline (speedup 1.0000x reference)
import functools

import jax
import jax.numpy as jnp
from jax import lax
from jax.experimental import pallas as pl
from jax.experimental.pallas import tpu as pltpu
from jax.experimental.pallas import tpu_sc as plsc

B = 16384
DIM = 64
ENT = 1000000
MARGIN = 1.0

NC, NS, L = 2, 16, 16
NW = NC * NS
ROWS_PER_W = B // NW
CHUNK = 128
NCHUNK = ROWS_PER_W // CHUNK
NG = CHUNK // L


def _sc_sqdist(ent_flat, rel_flat, ent_idx, rel_idx):
  mesh = plsc.VectorSubcoreMesh(core_axis_name="c", subcore_axis_name="s")

  @functools.partial(
      pl.kernel,
      mesh=mesh,
      compiler_params=pltpu.CompilerParams(
          needs_layout_passes=False, use_tc_tiling_on_sc=False),
      out_type=jax.ShapeDtypeStruct((2, B), jnp.float32),
      scratch_types=[
          pltpu.VMEM((6, ROWS_PER_W), jnp.int32),
          pltpu.VMEM((CHUNK,), jnp.int32),
          pltpu.VMEM((CHUNK,), jnp.int32),
          pltpu.VMEM((CHUNK,), jnp.int32),
          pltpu.VMEM((CHUNK,), jnp.float32),
          pltpu.VMEM((CHUNK,), jnp.float32),
          pltpu.VMEM((CHUNK,), jnp.float32),
          pltpu.VMEM((2, ROWS_PER_W), jnp.float32),
          pltpu.SemaphoreType.DMA,
      ],
  )
  def k(ent_idx_hbm, rel_idx_hbm, ent_hbm, rel_hbm, out_hbm,
        idx_all, idxph, idxpr, idxpt, h_v, r_v, t_v, ss_v, sem):
    wid = lax.axis_index("s") * NC + lax.axis_index("c")
    base = wid * ROWS_PER_W
    for side in range(2):
      pltpu.sync_copy(ent_idx_hbm.at[2 * side, pl.ds(base, ROWS_PER_W)],
                      idx_all.at[3 * side + 0])
      pltpu.sync_copy(rel_idx_hbm.at[side, pl.ds(base, ROWS_PER_W)],
                      idx_all.at[3 * side + 1])
      pltpu.sync_copy(ent_idx_hbm.at[2 * side + 1, pl.ds(base, ROWS_PER_W)],
                      idx_all.at[3 * side + 2])

    for side in range(2):
      for ck in range(NCHUNK):
        cb = ck * CHUNK

        def plane(f, accs):
          foff = f * ENT
          for role, idxp in ((0, idxph), (1, idxpr), (2, idxpt)):
            for g in range(NG):
              raw = idx_all[3 * side + role, pl.ds(cb + g * L, L)]
              idxp[pl.ds(g * L, L)] = raw + foff
          ch = pltpu.async_copy(ent_hbm.at[idxph], h_v, sem)
          cr = pltpu.async_copy(rel_hbm.at[idxpr], r_v, sem)
          ct = pltpu.async_copy(ent_hbm.at[idxpt], t_v, sem)
          ch.wait()
          cr.wait()
          ct.wait()
          new = []
          for g in range(NG):
            d = (h_v[pl.ds(g * L, L)] + r_v[pl.ds(g * L, L)]
                 - t_v[pl.ds(g * L, L)])
            new.append(accs[g] + d * d)
          return tuple(new)

        accs = lax.fori_loop(
            0, DIM, plane,
            tuple(jnp.zeros((L,), jnp.float32) for _ in range(NG)))
        for g in range(NG):
          ss_v[side, pl.ds(cb + g * L, L)] = accs[g]

    for side in range(2):
      pltpu.sync_copy(ss_v.at[side], out_hbm.at[side, pl.ds(base, ROWS_PER_W)])

  return k(ent_idx, rel_idx, ent_flat, rel_flat)


def _tc_loss(pos_ss, neg_ss):
  def body(p_ref, n_ref, o_ref):
    p = jnp.sqrt(p_ref[...])
    n = jnp.sqrt(n_ref[...])
    v = jnp.maximum(p - n + MARGIN, 0.0)
    o_ref[...] = (jnp.sum(v) * (1.0 / B)).reshape(1, 1)

  return pl.pallas_call(
      body,
      out_shape=jax.ShapeDtypeStruct((1, 1), jnp.float32),
  )(pos_ss, neg_ss)


def kernel(pos_triples, neg_triples, ent_emb, rel_emb):
  pt = pos_triples.astype(jnp.int32)
  nt = neg_triples.astype(jnp.int32)
  ent_idx = jnp.stack([pt[:, 0], pt[:, 2], nt[:, 0], nt[:, 2]])
  rel_idx = jnp.stack([pt[:, 1], nt[:, 1]])
  ent_flat = ent_emb.T.reshape(-1)
  rel_flat = rel_emb.T.reshape(-1)
  ss = _sc_sqdist(ent_flat, rel_flat, ent_idx, rel_idx)
  loss = _tc_loss(ss[0].reshape(128, 128), ss[1].reshape(128, 128))
  return loss[0, 0]


# speedup vs baseline: 3.6022x; 3.6022x over previous
import functools

import jax
import jax.numpy as jnp
from jax import lax
from jax.experimental import pallas as pl
from jax.experimental.pallas import tpu as pltpu
from jax.experimental.pallas import tpu_sc as plsc

B = 16384
DIM = 64
ENT = 1000000
MARGIN = 1.0

NC, NS, L = 2, 16, 16
NW = NC * NS
ROWS_PER_W = B // NW
CHUNK = 128
NCHUNK = ROWS_PER_W // CHUNK

CW = 512
NWIN = ENT // CW
TAIL = ENT - NWIN * CW
WPW = (NWIN + NW - 1) // NW


def _sc_detile(entT, relT):
  mesh = plsc.VectorSubcoreMesh(core_axis_name="c", subcore_axis_name="s")

  @functools.partial(
      pl.kernel,
      mesh=mesh,
      compiler_params=pltpu.CompilerParams(needs_layout_passes=False),
      out_type=[
          jax.ShapeDtypeStruct((ENT * DIM,), jnp.float32),
          jax.ShapeDtypeStruct((ENT * DIM,), jnp.float32),
      ],
      scratch_types=[
          pltpu.VMEM((DIM, CW), jnp.float32),
          pltpu.VMEM((CW * DIM,), jnp.float32),
          pltpu.VMEM((DIM, TAIL), jnp.float32),
          pltpu.VMEM((TAIL * DIM,), jnp.float32),
      ],
  )
  def k(entT_hbm, relT_hbm, ent_out, rel_out, in2d, ob, in2t, obt):
    wid = lax.axis_index("s") * NC + lax.axis_index("c")
    lane = jnp.arange(L, dtype=jnp.int32)
    for tab_hbm, out1d in ((entT_hbm, ent_out), (relT_hbm, rel_out)):

      def win(kk, _):
        w = wid + NW * kk

        @pl.when(w < NWIN)
        def _():
          c0 = pl.multiple_of(w * CW, CW)
          pltpu.sync_copy(tab_hbm.at[:, pl.ds(c0, CW)], in2d)

          def frow(f, _):
            for g in range(CW // L):
              v = in2d[f, pl.ds(g * L, L)]
              plsc.store_scatter(ob, [(lane + g * L) * DIM + f], v)
            return 0

          lax.fori_loop(0, DIM, frow, 0)
          pltpu.sync_copy(ob, out1d.at[pl.ds(c0 * DIM, CW * DIM)])

        return 0

      lax.fori_loop(0, WPW, win, 0)

    for t, (tab_hbm, out1d) in enumerate(((entT_hbm, ent_out),
                                          (relT_hbm, rel_out))):

      @pl.when(wid == t)
      def _():
        pltpu.sync_copy(tab_hbm.at[:, pl.ds(NWIN * CW, TAIL)], in2t)

        def frow(f, _):
          for g in range(TAIL // L):
            v = in2t[f, pl.ds(g * L, L)]
            plsc.store_scatter(obt, [(lane + g * L) * DIM + f], v)
          return 0

        lax.fori_loop(0, DIM, frow, 0)
        pltpu.sync_copy(obt, out1d.at[pl.ds(NWIN * CW * DIM, TAIL * DIM)])

  return k(entT, relT)


def _sc_sqdist(ent_idx, rel_idx, ent_emb, rel_emb):
  mesh = plsc.VectorSubcoreMesh(core_axis_name="c", subcore_axis_name="s")

  @functools.partial(
      pl.kernel,
      mesh=mesh,
      compiler_params=pltpu.CompilerParams(
          needs_layout_passes=False, use_tc_tiling_on_sc=False),
      out_type=jax.ShapeDtypeStruct((2, B), jnp.float32),
      scratch_types=[
          pltpu.VMEM((CHUNK,), jnp.int32),
          pltpu.VMEM((CHUNK,), jnp.int32),
          pltpu.VMEM((CHUNK,), jnp.int32),
          pltpu.VMEM((CHUNK, DIM), jnp.float32),
          pltpu.VMEM((CHUNK, DIM), jnp.float32),
          pltpu.VMEM((CHUNK, DIM), jnp.float32),
          pltpu.VMEM((L * L,), jnp.float32),
          pltpu.VMEM((CHUNK,), jnp.float32),
          pltpu.SemaphoreType.DMA,
      ],
  )
  def k(ent_idx_hbm, rel_idx_hbm, ent_hbm, rel_hbm, out_hbm,
        idxh_v, idxr_v, idxt_v, h_v, r_v, t_v, acc16_v, ss_v, sem):
    wid = lax.axis_index("s") * NC + lax.axis_index("c")
    base = wid * ROWS_PER_W
    iota = jnp.arange(L, dtype=jnp.int32)
    for side in range(2):
      for ck in range(NCHUNK):
        off = base + ck * CHUNK
        pltpu.sync_copy(ent_idx_hbm.at[2 * side, pl.ds(off, CHUNK)], idxh_v)
        pltpu.sync_copy(rel_idx_hbm.at[side, pl.ds(off, CHUNK)], idxr_v)
        pltpu.sync_copy(ent_idx_hbm.at[2 * side + 1, pl.ds(off, CHUNK)], idxt_v)
        ch = pltpu.async_copy(ent_hbm.at[idxh_v], h_v, sem)
        cr = pltpu.async_copy(rel_hbm.at[idxr_v], r_v, sem)
        ct = pltpu.async_copy(ent_hbm.at[idxt_v], t_v, sem)
        ch.wait()
        cr.wait()
        ct.wait()

        def group(g, _):
          rbase = pl.multiple_of(g * L, L)
          for r in range(L):
            i = rbase + r
            acc = jnp.zeros((L,), jnp.float32)
            for j in range(DIM // L):
              d = (h_v[i, pl.ds(j * L, L)] + r_v[i, pl.ds(j * L, L)]
                   - t_v[i, pl.ds(j * L, L)])
              acc = acc + d * d
            acc16_v[pl.ds(r * L, L)] = acc
          tot = jnp.zeros((L,), jnp.float32)
          for j in range(L):
            tot = tot + plsc.load_gather(acc16_v, [iota * L + j])
          ss_v[pl.ds(rbase, L)] = tot
          return 0

        lax.fori_loop(0, CHUNK // L, group, 0)
        pltpu.sync_copy(ss_v, out_hbm.at[side, pl.ds(off, CHUNK)])

  return k(ent_idx, rel_idx, ent_emb, rel_emb)


def _tc_loss(pos_ss, neg_ss):
  def body(p_ref, n_ref, o_ref):
    p = jnp.sqrt(p_ref[...])
    n = jnp.sqrt(n_ref[...])
    v = jnp.maximum(p - n + MARGIN, 0.0)
    o_ref[...] = (jnp.sum(v) * (1.0 / B)).reshape(1, 1)

  return pl.pallas_call(
      body,
      out_shape=jax.ShapeDtypeStruct((1, 1), jnp.float32),
  )(pos_ss, neg_ss)


def kernel(pos_triples, neg_triples, ent_emb, rel_emb):
  pt = pos_triples.astype(jnp.int32)
  nt = neg_triples.astype(jnp.int32)
  ent_idx = jnp.stack([pt[:, 0], pt[:, 2], nt[:, 0], nt[:, 2]])
  rel_idx = jnp.stack([pt[:, 1], nt[:, 1]])
  ent1d, rel1d = _sc_detile(ent_emb.T, rel_emb.T)
  ent_lin = ent1d.reshape(ENT, DIM)
  rel_lin = rel1d.reshape(ENT, DIM)
  ss = _sc_sqdist(ent_idx, rel_idx, ent_lin, rel_lin)
  loss = _tc_loss(ss[0].reshape(128, 128), ss[1].reshape(128, 128))
  return loss[0, 0]


# speedup vs baseline: 3.9830x; 1.1057x over previous
import functools

import jax
import jax.numpy as jnp
from jax import lax
from jax.experimental import pallas as pl
from jax.experimental.pallas import tpu as pltpu
from jax.experimental.pallas import tpu_sc as plsc

B = 16384
DIM = 64
ENT = 1000000
MARGIN = 1.0

NC, NS, L = 2, 16, 16
NW = NC * NS
ROWS_PER_W = B // NW
CHUNK = 128
NCHUNK = ROWS_PER_W // CHUNK

CW = 384
NWIN = ENT // CW
TAIL = ENT - NWIN * CW
WPW = (NWIN + NW - 1) // NW


def _sc_detile(entT, relT):
  mesh = plsc.VectorSubcoreMesh(core_axis_name="c", subcore_axis_name="s")

  @functools.partial(
      pl.kernel,
      mesh=mesh,
      compiler_params=pltpu.CompilerParams(needs_layout_passes=False),
      out_type=[
          jax.ShapeDtypeStruct((ENT * DIM,), jnp.float32),
          jax.ShapeDtypeStruct((ENT * DIM,), jnp.float32),
      ],
      scratch_types=[
          pltpu.VMEM((DIM, CW), jnp.float32),
          pltpu.VMEM((DIM, CW), jnp.float32),
          pltpu.VMEM((CW * DIM,), jnp.float32),
          pltpu.VMEM((DIM, TAIL), jnp.float32),
          pltpu.SemaphoreType.DMA,
      ],
  )
  def k(entT_hbm, relT_hbm, ent_out, rel_out, in_a, in_b, ob, in2t, semi):
    wid = lax.axis_index("s") * NC + lax.axis_index("c")
    lane = jnp.arange(L, dtype=jnp.int32)
    bufs = (in_a, in_b)

    def pump(inp, ncols):
      def frow(fi, _):
        for u in range(4):
          f = fi * 4 + u
          for g in range(ncols // L):
            v = inp[f, pl.ds(g * L, L)]
            plsc.store_scatter(ob, [(lane + g * L) * DIM + f], v)
        return 0

      lax.fori_loop(0, DIM // 4, frow, 0)

    for tab_hbm, out1d in ((entT_hbm, ent_out), (relT_hbm, rel_out)):
      pltpu.async_copy(
          tab_hbm.at[:, pl.ds(pl.multiple_of(wid * CW, 128), CW)],
          in_a, semi)

      def win2(kk, _):
        for u in range(2):
          k2 = kk * 2 + u
          w = wid + NW * k2
          wn = wid + NW * (k2 + 1)

          @pl.when(wn < NWIN)
          def _():
            pltpu.async_copy(
                tab_hbm.at[:, pl.ds(pl.multiple_of(wn * CW, 128), CW)],
                bufs[1 - u], semi)

          @pl.when(w < NWIN)
          def _():
            pltpu.make_async_copy(
                tab_hbm.at[:, pl.ds(0, CW)], bufs[u], semi).wait()
            pump(bufs[u], CW)
            pltpu.sync_copy(
                ob, out1d.at[pl.ds(pl.multiple_of(w * (CW * DIM), 8),
                                   CW * DIM)])

        return 0

      lax.fori_loop(0, WPW // 2, win2, 0)

    for t, (tab_hbm, out1d) in enumerate(((entT_hbm, ent_out),
                                          (relT_hbm, rel_out))):

      @pl.when(wid == t)
      def _():
        pltpu.sync_copy(tab_hbm.at[:, pl.ds(NWIN * CW, TAIL)], in2t)

        def frowt(f, _):
          for g in range(TAIL // L):
            v = in2t[f, pl.ds(g * L, L)]
            plsc.store_scatter(ob, [(lane + g * L) * DIM + f], v)
          return 0

        lax.fori_loop(0, DIM, frowt, 0)
        pltpu.sync_copy(ob.at[pl.ds(0, TAIL * DIM)],
                        out1d.at[pl.ds(NWIN * CW * DIM, TAIL * DIM)])

  return k(entT, relT)


def _sc_sqdist(ent_idx, rel_idx, ent_emb, rel_emb):
  mesh = plsc.VectorSubcoreMesh(core_axis_name="c", subcore_axis_name="s")

  @functools.partial(
      pl.kernel,
      mesh=mesh,
      compiler_params=pltpu.CompilerParams(
          needs_layout_passes=False, use_tc_tiling_on_sc=False),
      out_type=jax.ShapeDtypeStruct((2, B), jnp.float32),
      scratch_types=[
          pltpu.VMEM((CHUNK,), jnp.int32),
          pltpu.VMEM((CHUNK,), jnp.int32),
          pltpu.VMEM((CHUNK,), jnp.int32),
          pltpu.VMEM((CHUNK, DIM), jnp.float32),
          pltpu.VMEM((CHUNK, DIM), jnp.float32),
          pltpu.VMEM((CHUNK, DIM), jnp.float32),
          pltpu.VMEM((L * L,), jnp.float32),
          pltpu.VMEM((CHUNK,), jnp.float32),
          pltpu.SemaphoreType.DMA,
      ],
  )
  def k(ent_idx_hbm, rel_idx_hbm, ent_hbm, rel_hbm, out_hbm,
        idxh_v, idxr_v, idxt_v, h_v, r_v, t_v, acc16_v, ss_v, sem):
    wid = lax.axis_index("s") * NC + lax.axis_index("c")
    base = wid * ROWS_PER_W
    iota = jnp.arange(L, dtype=jnp.int32)
    for side in range(2):
      for ck in range(NCHUNK):
        off = base + ck * CHUNK
        pltpu.sync_copy(ent_idx_hbm.at[2 * side, pl.ds(off, CHUNK)], idxh_v)
        pltpu.sync_copy(rel_idx_hbm.at[side, pl.ds(off, CHUNK)], idxr_v)
        pltpu.sync_copy(ent_idx_hbm.at[2 * side + 1, pl.ds(off, CHUNK)], idxt_v)
        ch = pltpu.async_copy(ent_hbm.at[idxh_v], h_v, sem)
        cr = pltpu.async_copy(rel_hbm.at[idxr_v], r_v, sem)
        ct = pltpu.async_copy(ent_hbm.at[idxt_v], t_v, sem)
        ch.wait()
        cr.wait()
        ct.wait()

        def group(g, _):
          rbase = pl.multiple_of(g * L, L)
          for r in range(L):
            i = rbase + r
            acc = jnp.zeros((L,), jnp.float32)
            for j in range(DIM // L):
              d = (h_v[i, pl.ds(j * L, L)] + r_v[i, pl.ds(j * L, L)]
                   - t_v[i, pl.ds(j * L, L)])
              acc = acc + d * d
            acc16_v[pl.ds(r * L, L)] = acc
          tot = jnp.zeros((L,), jnp.float32)
          for j in range(L):
            tot = tot + plsc.load_gather(acc16_v, [iota * L + j])
          ss_v[pl.ds(rbase, L)] = tot
          return 0

        lax.fori_loop(0, CHUNK // L, group, 0)
        pltpu.sync_copy(ss_v, out_hbm.at[side, pl.ds(off, CHUNK)])

  return k(ent_idx, rel_idx, ent_emb, rel_emb)


def _tc_loss(pos_ss, neg_ss):
  def body(p_ref, n_ref, o_ref):
    p = jnp.sqrt(p_ref[...])
    n = jnp.sqrt(n_ref[...])
    v = jnp.maximum(p - n + MARGIN, 0.0)
    o_ref[...] = (jnp.sum(v) * (1.0 / B)).reshape(1, 1)

  return pl.pallas_call(
      body,
      out_shape=jax.ShapeDtypeStruct((1, 1), jnp.float32),
  )(pos_ss, neg_ss)


def kernel(pos_triples, neg_triples, ent_emb, rel_emb):
  pt = pos_triples.astype(jnp.int32)
  nt = neg_triples.astype(jnp.int32)
  ent_idx = jnp.stack([pt[:, 0], pt[:, 2], nt[:, 0], nt[:, 2]])
  rel_idx = jnp.stack([pt[:, 1], nt[:, 1]])
  ent1d, rel1d = _sc_detile(ent_emb.T, rel_emb.T)
  ent_lin = ent1d.reshape(ENT, DIM)
  rel_lin = rel1d.reshape(ENT, DIM)
  ss = _sc_sqdist(ent_idx, rel_idx, ent_lin, rel_lin)
  loss = _tc_loss(ss[0].reshape(128, 128), ss[1].reshape(128, 128))
  return loss[0, 0]


# speedup vs baseline: 9.2605x; 2.3250x over previous
import functools

import jax
import jax.numpy as jnp
from jax import lax
from jax.experimental import pallas as pl
from jax.experimental.pallas import tpu as pltpu
from jax.experimental.pallas import tpu_sc as plsc

B = 16384
DIM = 64
MARGIN = 1.0

NC, NS, L = 2, 16, 16
NW = NC * NS
ROWS_PER_W = B // NW
CHUNK = 128
NCHUNK = ROWS_PER_W // CHUNK


def _sc_sqdist(ent_idx, rel_idx, ent_emb, rel_emb):
  mesh = plsc.VectorSubcoreMesh(core_axis_name="c", subcore_axis_name="s")

  @functools.partial(
      pl.kernel,
      mesh=mesh,
      compiler_params=pltpu.CompilerParams(
          needs_layout_passes=False, use_tc_tiling_on_sc=False),
      out_type=jax.ShapeDtypeStruct((2, B), jnp.float32),
      scratch_types=[
          pltpu.VMEM((CHUNK,), jnp.int32),
          pltpu.VMEM((CHUNK,), jnp.int32),
          pltpu.VMEM((CHUNK,), jnp.int32),
          pltpu.VMEM((CHUNK, DIM), jnp.float32),
          pltpu.VMEM((CHUNK, DIM), jnp.float32),
          pltpu.VMEM((CHUNK, DIM), jnp.float32),
          pltpu.VMEM((L * L,), jnp.float32),
          pltpu.VMEM((CHUNK,), jnp.float32),
          pltpu.SemaphoreType.DMA,
      ],
  )
  def k(ent_idx_hbm, rel_idx_hbm, ent_hbm, rel_hbm, out_hbm,
        idxh_v, idxr_v, idxt_v, h_v, r_v, t_v, acc16_v, ss_v, sem):
    wid = lax.axis_index("s") * NC + lax.axis_index("c")
    base = wid * ROWS_PER_W
    iota = jnp.arange(L, dtype=jnp.int32)
    for side in range(2):
      for ck in range(NCHUNK):
        off = base + ck * CHUNK
        pltpu.sync_copy(ent_idx_hbm.at[2 * side, pl.ds(off, CHUNK)], idxh_v)
        pltpu.sync_copy(rel_idx_hbm.at[side, pl.ds(off, CHUNK)], idxr_v)
        pltpu.sync_copy(ent_idx_hbm.at[2 * side + 1, pl.ds(off, CHUNK)], idxt_v)
        ch = pltpu.async_copy(ent_hbm.at[idxh_v], h_v, sem)
        cr = pltpu.async_copy(rel_hbm.at[idxr_v], r_v, sem)
        ct = pltpu.async_copy(ent_hbm.at[idxt_v], t_v, sem)
        ch.wait()
        cr.wait()
        ct.wait()

        def group(g, _):
          rbase = pl.multiple_of(g * L, L)
          for r in range(L):
            i = rbase + r
            acc = jnp.zeros((L,), jnp.float32)
            for j in range(DIM // L):
              d = (h_v[i, pl.ds(j * L, L)] + r_v[i, pl.ds(j * L, L)]
                   - t_v[i, pl.ds(j * L, L)])
              acc = acc + d * d
            acc16_v[pl.ds(r * L, L)] = acc
          tot = jnp.zeros((L,), jnp.float32)
          for j in range(L):
            tot = tot + plsc.load_gather(acc16_v, [iota * L + j])
          ss_v[pl.ds(rbase, L)] = tot
          return 0

        lax.fori_loop(0, CHUNK // L, group, 0)
        pltpu.sync_copy(ss_v, out_hbm.at[side, pl.ds(off, CHUNK)])

  return k(ent_idx, rel_idx, ent_emb, rel_emb)


def _tc_loss(pos_ss, neg_ss):
  def body(p_ref, n_ref, o_ref):
    p = jnp.sqrt(p_ref[...])
    n = jnp.sqrt(n_ref[...])
    v = jnp.maximum(p - n + MARGIN, 0.0)
    o_ref[...] = (jnp.sum(v) * (1.0 / B)).reshape(1, 1)

  return pl.pallas_call(
      body,
      out_shape=jax.ShapeDtypeStruct((1, 1), jnp.float32),
  )(pos_ss, neg_ss)


def kernel(pos_triples, neg_triples, ent_emb, rel_emb):
  pt = pos_triples.astype(jnp.int32)
  nt = neg_triples.astype(jnp.int32)
  ent_idx = jnp.stack([pt[:, 0], pt[:, 2], nt[:, 0], nt[:, 2]])
  rel_idx = jnp.stack([pt[:, 1], nt[:, 1]])
  ss = _sc_sqdist(ent_idx, rel_idx, ent_emb, rel_emb)
  loss = _tc_loss(ss[0].reshape(128, 128), ss[1].reshape(128, 128))
  return loss[0, 0]


# speedup vs baseline: 9.3293x; 1.0074x over previous
import functools

import jax
import jax.numpy as jnp
from jax import lax
from jax.experimental import pallas as pl
from jax.experimental.pallas import tpu as pltpu
from jax.experimental.pallas import tpu_sc as plsc

B = 16384
DIM = 64
MARGIN = 1.0

NC, NS, L = 2, 16, 16
NW = NC * NS
ROWS_PER_W = B // NW
CHUNK = 128
NCHUNK = ROWS_PER_W // CHUNK


def _sc_hmt(ent_idx, ent_emb):
  mesh = plsc.VectorSubcoreMesh(core_axis_name="c", subcore_axis_name="s")

  @functools.partial(
      pl.kernel,
      mesh=mesh,
      compiler_params=pltpu.CompilerParams(
          needs_layout_passes=False, use_tc_tiling_on_sc=False),
      out_type=jax.ShapeDtypeStruct((2, B, DIM), jnp.float32),
      scratch_types=[
          pltpu.VMEM((CHUNK,), jnp.int32),
          pltpu.VMEM((CHUNK,), jnp.int32),
          pltpu.VMEM((CHUNK, DIM), jnp.float32),
          pltpu.VMEM((CHUNK, DIM), jnp.float32),
          pltpu.SemaphoreType.DMA,
      ],
  )
  def k(ent_idx_hbm, ent_hbm, out_hbm, idxh_v, idxt_v, h_v, t_v, sem):
    wid = lax.axis_index("s") * NC + lax.axis_index("c")
    base = wid * ROWS_PER_W
    for side in range(2):
      for ck in range(NCHUNK):
        off = base + ck * CHUNK
        pltpu.sync_copy(ent_idx_hbm.at[2 * side, pl.ds(off, CHUNK)], idxh_v)
        pltpu.sync_copy(ent_idx_hbm.at[2 * side + 1, pl.ds(off, CHUNK)], idxt_v)
        ch = pltpu.async_copy(ent_hbm.at[idxh_v], h_v, sem)
        ct = pltpu.async_copy(ent_hbm.at[idxt_v], t_v, sem)
        ch.wait()
        ct.wait()

        def row(i, _):
          for j in range(DIM // L):
            h_v[i, pl.ds(j * L, L)] = (h_v[i, pl.ds(j * L, L)]
                                       - t_v[i, pl.ds(j * L, L)])
          return 0

        lax.fori_loop(0, CHUNK, row, 0)
        pltpu.sync_copy(h_v, out_hbm.at[side, pl.ds(off, CHUNK), :])

  return k(ent_idx, ent_emb)


def _sc_sqdist(rel_idx, s_rows, rel_emb):
  mesh = plsc.VectorSubcoreMesh(core_axis_name="c", subcore_axis_name="s")

  @functools.partial(
      pl.kernel,
      mesh=mesh,
      compiler_params=pltpu.CompilerParams(
          needs_layout_passes=False, use_tc_tiling_on_sc=False),
      out_type=jax.ShapeDtypeStruct((2, B), jnp.float32),
      scratch_types=[
          pltpu.VMEM((CHUNK,), jnp.int32),
          pltpu.VMEM((CHUNK, DIM), jnp.float32),
          pltpu.VMEM((CHUNK, DIM), jnp.float32),
          pltpu.VMEM((L * L,), jnp.float32),
          pltpu.VMEM((CHUNK,), jnp.float32),
          pltpu.SemaphoreType.DMA,
      ],
  )
  def k(rel_idx_hbm, s_hbm, rel_hbm, out_hbm,
        idxr_v, s_v, r_v, acc16_v, ss_v, sem):
    wid = lax.axis_index("s") * NC + lax.axis_index("c")
    base = wid * ROWS_PER_W
    iota = jnp.arange(L, dtype=jnp.int32)
    for side in range(2):
      for ck in range(NCHUNK):
        off = base + ck * CHUNK
        pltpu.sync_copy(rel_idx_hbm.at[side, pl.ds(off, CHUNK)], idxr_v)
        cs = pltpu.async_copy(s_hbm.at[side, pl.ds(off, CHUNK), :], s_v, sem)
        cr = pltpu.async_copy(rel_hbm.at[idxr_v], r_v, sem)
        cs.wait()
        cr.wait()

        def group(g, _):
          rbase = pl.multiple_of(g * L, L)
          for r in range(L):
            i = rbase + r
            acc = jnp.zeros((L,), jnp.float32)
            for j in range(DIM // L):
              d = s_v[i, pl.ds(j * L, L)] + r_v[i, pl.ds(j * L, L)]
              acc = acc + d * d
            acc16_v[pl.ds(r * L, L)] = acc
          tot = jnp.zeros((L,), jnp.float32)
          for j in range(L):
            tot = tot + plsc.load_gather(acc16_v, [iota * L + j])
          ss_v[pl.ds(rbase, L)] = tot
          return 0

        lax.fori_loop(0, CHUNK // L, group, 0)
        pltpu.sync_copy(ss_v, out_hbm.at[side, pl.ds(off, CHUNK)])

  return k(rel_idx, s_rows, rel_emb)


def _tc_loss(pos_ss, neg_ss):
  def body(p_ref, n_ref, o_ref):
    p = jnp.sqrt(p_ref[...])
    n = jnp.sqrt(n_ref[...])
    v = jnp.maximum(p - n + MARGIN, 0.0)
    o_ref[...] = (jnp.sum(v) * (1.0 / B)).reshape(1, 1)

  return pl.pallas_call(
      body,
      out_shape=jax.ShapeDtypeStruct((1, 1), jnp.float32),
  )(pos_ss, neg_ss)


def kernel(pos_triples, neg_triples, ent_emb, rel_emb):
  pt = pos_triples.astype(jnp.int32)
  nt = neg_triples.astype(jnp.int32)
  ent_idx = jnp.stack([pt[:, 0], pt[:, 2], nt[:, 0], nt[:, 2]])
  rel_idx = jnp.stack([pt[:, 1], nt[:, 1]])
  s_rows = _sc_hmt(ent_idx, ent_emb)
  ss = _sc_sqdist(rel_idx, s_rows, rel_emb)
  loss = _tc_loss(ss[0].reshape(128, 128), ss[1].reshape(128, 128))
  return loss[0, 0]
